# Initial kernel scaffold; baseline (speedup 1.0000x reference)
import jax
import jax.numpy as jnp
from jax.experimental import pallas as pl


def kernel(x, seg, tok_table, pos_table, seg_table, gamma, beta):
    raise NotImplementedError("write your pallas kernel here")


# speedup vs baseline: 1.0252x; 1.0252x over previous
import functools

import jax
import jax.numpy as jnp
from jax import lax
from jax.experimental import pallas as pl
from jax.experimental.pallas import tpu as pltpu
from jax.experimental.pallas import tpu_sc as plsc

B = 1024
L = 50
D = 768
N = B * L
NLANES = 16
NJ = D // NLANES
NC = 2
NS = 16
NW = NC * NS
TOK_PER_W = N // NW
CHUNK = 40
NCHUNK = TOK_PER_W // CHUNK
EPS = 1e-5


def _rsqrt_newton(v):
    i = lax.bitcast_convert_type(v, jnp.int32)
    i = jnp.full_like(i, 0x5F3759DF) - lax.shift_right_arithmetic(i, 1)
    y = lax.bitcast_convert_type(i, jnp.float32)
    for _ in range(3):
        y = y * (1.5 - 0.5 * v * y * y)
    return y


_GATHER_DN = lax.GatherDimensionNumbers(
    offset_dims=(), collapsed_slice_dims=(0,), start_index_map=(0,))


def _shuffle(v, idx):
    return lax.gather(v, idx[:, None], _GATHER_DN, (1,),
                      mode=lax.GatherScatterMode.PROMISE_IN_BOUNDS)


def _lane_sum(v):
    lanes = lax.iota(jnp.int32, NLANES)
    for k in (8, 4, 2, 1):
        v = v + _shuffle(v, lanes ^ k)
    return v


def _sc_body(x_hbm, cidx_hbm, comb_hbm, tok_hbm, gamma_hbm, beta_hbm,
             out_hbm, comb_v, buf, idx_v, cidx_v, gamma_v, beta_v, sem):
    wid = lax.axis_index("s") * NC + lax.axis_index("c")
    base = wid * TOK_PER_W

    pltpu.sync_copy(comb_hbm, comb_v)
    pltpu.sync_copy(gamma_hbm, gamma_v)
    pltpu.sync_copy(beta_hbm, beta_v)
    pltpu.sync_copy(x_hbm.at[pl.ds(base, TOK_PER_W)], idx_v)
    pltpu.sync_copy(cidx_hbm.at[pl.ds(base, TOK_PER_W)],
                    cidx_v.at[pl.ds(0, TOK_PER_W)])

    def chunk_body(c, carry):
        t0 = c * CHUNK
        pltpu.async_copy(tok_hbm.at[idx_v.at[pl.ds(t0, CHUNK)]], buf,
                         sem).wait()

        def tok_body(r, carry2):
            crow = cidx_v[pl.ds(t0 + r, NLANES)][0]
            acc_s = jnp.zeros((NLANES,), jnp.float32)
            acc_q = jnp.zeros((NLANES,), jnp.float32)
            for j in range(NJ):
                sl = pl.ds(j * NLANES, NLANES)
                h = buf[r, sl] + comb_v[crow, sl]
                buf[r, sl] = h
                acc_s = acc_s + h
                acc_q = acc_q + h * h
            s = _lane_sum(acc_s)
            q = _lane_sum(acc_q)
            mu = s * (1.0 / D)
            var = q * (1.0 / D) - mu * mu
            rstd = _rsqrt_newton(var + EPS)
            for j in range(NJ):
                sl = pl.ds(j * NLANES, NLANES)
                h = buf[r, sl]
                buf[r, sl] = (h - mu) * (rstd * gamma_v[sl]) + beta_v[sl]
            return carry2

        lax.fori_loop(0, CHUNK, tok_body, 0)
        pltpu.sync_copy(buf, out_hbm.at[pl.ds(base + t0, CHUNK)])
        return carry

    lax.fori_loop(0, NCHUNK, chunk_body, 0)


def kernel(x, seg, tok_table, pos_table, seg_table, gamma, beta):
    pos50 = pos_table[:L]
    comb = jnp.concatenate([pos50 + seg_table[0], pos50 + seg_table[1]], axis=0)
    cidx = (seg * L + jnp.arange(L, dtype=jnp.int32)[None, :]).reshape(N)
    xf = x.reshape(N)

    mesh = plsc.VectorSubcoreMesh(core_axis_name="c", subcore_axis_name="s")
    f = functools.partial(
        pl.kernel,
        mesh=mesh,
        out_type=jax.ShapeDtypeStruct((N, D), jnp.float32),
        scratch_types=[
            pltpu.VMEM((2 * L, D), jnp.float32),
            pltpu.VMEM((CHUNK, D), jnp.float32),
            pltpu.VMEM((TOK_PER_W,), jnp.int32),
            pltpu.VMEM((TOK_PER_W + NLANES,), jnp.int32),
            pltpu.VMEM((D,), jnp.float32),
            pltpu.VMEM((D,), jnp.float32),
            pltpu.SemaphoreType.DMA,
        ],
    )(_sc_body)
    return f(xf, cidx, comb, tok_table, gamma, beta).reshape(B, L, D)
